# Initial kernel scaffold; baseline (speedup 1.0000x reference)
import jax
import jax.numpy as jnp
from jax.experimental import pallas as pl


def kernel(x_num, candidate_x_num, candidate_y, params, context_size):
    raise NotImplementedError("write your pallas kernel here")


# speedup vs baseline: 1.0193x; 1.0193x over previous
import functools

import jax
import jax.numpy as jnp
from jax.experimental import pallas as pl
from jax.experimental.pallas import tpu as pltpu

D_IN = 128
D_MAIN = 256
D_BLOCK = 512
CTX = 32


def _dot(a, b):
    return jax.lax.dot_general(
        a.astype(jnp.bfloat16), b.astype(jnp.bfloat16),
        (((1,), (1,)), ((), ())), preferred_element_type=jnp.float32)


def _ln(x, g, b):
    m = jnp.mean(x, axis=-1, keepdims=True)
    d = x - m
    v = jnp.mean(d * d, axis=-1, keepdims=True)
    return d / jnp.sqrt(v + 1e-5) * g + b


def _enc_body(write_x, xin_ref, wlin_ref, blin_ref, w0a_ref, b0a_ref,
              w0b_ref, b0b_ref, g1_ref, be1_ref, w1a_ref, b1a_ref,
              w1b_ref, b1b_ref, gmix_ref, bmix_ref, wk_ref, bk_ref,
              *out_refs):
    dot = _dot
    x = dot(xin_ref[...], wlin_ref[...]) + blin_ref[...]
    h = dot(jnp.maximum(dot(x, w0a_ref[...]) + b0a_ref[...], 0.0),
            w0b_ref[...]) + b0b_ref[...]
    x = x + h
    z = _ln(x, g1_ref[...], be1_ref[...])
    h = dot(jnp.maximum(dot(z, w1a_ref[...]) + b1a_ref[...], 0.0),
            w1b_ref[...]) + b1b_ref[...]
    x = x + h
    z2 = _ln(x, gmix_ref[...], bmix_ref[...])
    k = dot(z2, wk_ref[...]) + bk_ref[...]
    if write_x:
        out_refs[0][...] = x
        out_refs[1][...] = k
    else:
        out_refs[0][...] = k


def _encode(xin, p, write_x, bm):
    M = xin.shape[0]
    grid = (M // bm,)
    row = lambda i: (i, 0)
    fixed = lambda i: (0, 0)
    w_specs = []
    w_args = []
    for w, b in ((p['W_lin'], p['b_lin']), (p['W0a'], p['b0a']),
                 (p['W0b'], p['b0b'])):
        w_args += [w, b.reshape(1, -1)]
    w_args += [p['g1'].reshape(1, -1), p['be1'].reshape(1, -1)]
    for w, b in ((p['W1a'], p['b1a']), (p['W1b'], p['b1b'])):
        w_args += [w, b.reshape(1, -1)]
    w_args += [p['g_mix'].reshape(1, -1), p['b_mix'].reshape(1, -1),
               p['WK'], p['bK'].reshape(1, -1)]
    in_specs = [pl.BlockSpec((bm, D_IN), row)]
    for a in w_args:
        in_specs.append(pl.BlockSpec(a.shape, fixed))
    if write_x:
        out_shape = [jax.ShapeDtypeStruct((M, D_MAIN), jnp.float32),
                     jax.ShapeDtypeStruct((M, D_MAIN), jnp.float32)]
        out_specs = [pl.BlockSpec((bm, D_MAIN), row),
                     pl.BlockSpec((bm, D_MAIN), row)]
    else:
        out_shape = jax.ShapeDtypeStruct((M, D_MAIN), jnp.float32)
        out_specs = pl.BlockSpec((bm, D_MAIN), row)
    return pl.pallas_call(
        functools.partial(_enc_body, write_x),
        grid=grid,
        in_specs=in_specs,
        out_specs=out_specs,
        out_shape=out_shape,
    )(xin, *w_args)


def _sims_body(kq_ref, ck_ref, out_ref):
    ck = ck_ref[...]
    csq = jnp.sum(ck * ck, axis=1)
    d = _dot(kq_ref[...], ck)
    out_ref[...] = 2.0 * d - csq[None, :]


def _sims(kq, ck, bq, bc):
    B = kq.shape[0]
    N = ck.shape[0]
    return pl.pallas_call(
        _sims_body,
        grid=(B // bq, N // bc),
        in_specs=[pl.BlockSpec((bq, D_MAIN), lambda i, j: (i, 0)),
                  pl.BlockSpec((bc, D_MAIN), lambda i, j: (j, 0))],
        out_specs=pl.BlockSpec((bq, bc), lambda i, j: (i, j)),
        out_shape=jax.ShapeDtypeStruct((B, N), jnp.float32),
    )(kq, ck)


def _tail_body(vals_ref, ctxk_ref, ctxy_ref, kq_ref, x_ref,
               wle_ref, ble_ref, wta_ref, bta_ref, wtb_ref,
               ga_ref, ba_ref, waa_ref, baa_ref, wab_ref, bab_ref,
               gb_ref, bb_ref, wba_ref, bba_ref, wbb_ref, bbb_ref,
               gh_ref, bh_ref, wh_ref, bhl_ref, out_ref):
    dot = _dot
    vals = vals_ref[...]
    m = jnp.max(vals, axis=-1, keepdims=True)
    e = jnp.exp(vals - m)
    probs = e / jnp.sum(e, axis=-1, keepdims=True)
    bq = vals.shape[0]
    kq = kq_ref[...]
    ctxk = ctxk_ref[...].reshape(bq, CTX, D_MAIN)
    u = (kq[:, None, :] - ctxk).reshape(bq * CTX, D_MAIN)
    t = dot(jnp.maximum(dot(u, wta_ref[...]) + bta_ref[...], 0.0),
            wtb_ref[...]).reshape(bq, CTX, D_MAIN)
    wle_b = jnp.broadcast_to(wle_ref[...], (bq, D_MAIN))
    ble_b = jnp.broadcast_to(ble_ref[...], (bq, D_MAIN))
    ctxy = ctxy_ref[...]
    probs_b = probs.astype(jnp.bfloat16).astype(jnp.float32)
    acc = jnp.zeros((bq, D_MAIN), jnp.float32)
    for c in range(CTX):
        v_c = ctxy[:, c:c + 1] * wle_b + ble_b + t[:, c, :]
        v_c = v_c.astype(jnp.bfloat16).astype(jnp.float32)
        acc = acc + probs_b[:, c:c + 1] * v_c
    x = x_ref[...] + acc
    for g, b, wa, ba, wb, bb in (
            (ga_ref, ba_ref, waa_ref, baa_ref, wab_ref, bab_ref),
            (gb_ref, bb_ref, wba_ref, bba_ref, wbb_ref, bbb_ref)):
        z = _ln(x, g[...], b[...])
        h = dot(jnp.maximum(dot(z, wa[...]) + ba[...], 0.0),
                wb[...]) + bb[...]
        x = x + h
    z = _ln(x, gh_ref[...], bh_ref[...])
    out_ref[...] = dot(jnp.maximum(z, 0.0), wh_ref[...]) + bhl_ref[...]


def _tail_head_pad(p):
    wh = jnp.zeros((128, D_MAIN), jnp.float32).at[0].set(p['WH'][0])
    bhl = jnp.zeros((1, 128), jnp.float32).at[0, 0].set(p['bH_lin'][0])
    return wh, bhl


def _tail(vals, ctxk2d, ctxy, kq, x, p, bq):
    B = vals.shape[0]
    row = lambda i: (i, 0)
    fixed = lambda i: (0, 0)
    wh_pad, bhl_pad = _tail_head_pad(p)
    w_args = [p['W_le'].reshape(1, D_MAIN), p['b_le'].reshape(1, -1),
              p['WTa'], p['bTa'].reshape(1, -1), p['WTb'],
              p['gA'].reshape(1, -1), p['bA'].reshape(1, -1),
              p['WAa'], p['bAa'].reshape(1, -1), p['WAb'], p['bAb'].reshape(1, -1),
              p['gB'].reshape(1, -1), p['bB'].reshape(1, -1),
              p['WBa'], p['bBa'].reshape(1, -1), p['WBb'], p['bBb'].reshape(1, -1),
              p['gH'].reshape(1, -1), p['bH'].reshape(1, -1),
              wh_pad, bhl_pad]
    in_specs = [pl.BlockSpec((bq, CTX), row),
                pl.BlockSpec((bq * CTX, D_MAIN), row),
                pl.BlockSpec((bq, CTX), row),
                pl.BlockSpec((bq, D_MAIN), row),
                pl.BlockSpec((bq, D_MAIN), row)]
    for a in w_args:
        in_specs.append(pl.BlockSpec(a.shape, fixed))
    return pl.pallas_call(
        _tail_body,
        grid=(B // bq,),
        in_specs=in_specs,
        out_specs=pl.BlockSpec((bq, 128), row),
        out_shape=jax.ShapeDtypeStruct((B, 128), jnp.float32),
    )(vals, ctxk2d, ctxy, kq, x, *w_args)[:, :1]


def kernel(x_num, candidate_x_num, candidate_y, params, context_size):
    p = params
    B = x_num.shape[0]
    N = candidate_x_num.shape[0]
    ck = _encode(candidate_x_num, p, write_x=False, bm=2048)
    xq, kq = _encode(x_num, p, write_x=True, bm=1024)
    sims = _sims(kq, ck, bq=256, bc=2048)
    vals, idx = jax.lax.top_k(sims, CTX)
    ctxk = ck[idx]
    ctxy = candidate_y[idx]
    out = _tail(vals, ctxk.reshape(B * CTX, D_MAIN), ctxy, kq, xq, p, bq=256)
    return out


# speedup vs baseline: 2.4163x; 2.3705x over previous
import functools

import jax
import jax.numpy as jnp
from jax import lax
from jax.experimental import pallas as pl
from jax.experimental.pallas import tpu as pltpu
from jax.experimental.pallas import tpu_sc as plsc

D_IN = 128
D_MAIN = 256
D_BLOCK = 512
CTX = 32


def _dot(a, b):
    return jax.lax.dot_general(
        a.astype(jnp.bfloat16), b.astype(jnp.bfloat16),
        (((1,), (1,)), ((), ())), preferred_element_type=jnp.float32)


def _ln(x, g, b):
    m = jnp.mean(x, axis=-1, keepdims=True)
    d = x - m
    v = jnp.mean(d * d, axis=-1, keepdims=True)
    return d / jnp.sqrt(v + 1e-5) * g + b


def _enc_body(write_x, xin_ref, wlin_ref, blin_ref, w0a_ref, b0a_ref,
              w0b_ref, b0b_ref, g1_ref, be1_ref, w1a_ref, b1a_ref,
              w1b_ref, b1b_ref, gmix_ref, bmix_ref, wk_ref, bk_ref,
              *out_refs):
    dot = _dot
    x = dot(xin_ref[...], wlin_ref[...]) + blin_ref[...]
    h = dot(jnp.maximum(dot(x, w0a_ref[...]) + b0a_ref[...], 0.0),
            w0b_ref[...]) + b0b_ref[...]
    x = x + h
    z = _ln(x, g1_ref[...], be1_ref[...])
    h = dot(jnp.maximum(dot(z, w1a_ref[...]) + b1a_ref[...], 0.0),
            w1b_ref[...]) + b1b_ref[...]
    x = x + h
    z2 = _ln(x, gmix_ref[...], bmix_ref[...])
    k = dot(z2, wk_ref[...]) + bk_ref[...]
    if write_x:
        out_refs[0][...] = x
        out_refs[1][...] = k
    else:
        out_refs[0][...] = k


def _encode(xin, p, write_x, bm):
    M = xin.shape[0]
    grid = (M // bm,)
    row = lambda i: (i, 0)
    fixed = lambda i: (0, 0)
    w_specs = []
    w_args = []
    for w, b in ((p['W_lin'], p['b_lin']), (p['W0a'], p['b0a']),
                 (p['W0b'], p['b0b'])):
        w_args += [w, b.reshape(1, -1)]
    w_args += [p['g1'].reshape(1, -1), p['be1'].reshape(1, -1)]
    for w, b in ((p['W1a'], p['b1a']), (p['W1b'], p['b1b'])):
        w_args += [w, b.reshape(1, -1)]
    w_args += [p['g_mix'].reshape(1, -1), p['b_mix'].reshape(1, -1),
               p['WK'], p['bK'].reshape(1, -1)]
    in_specs = [pl.BlockSpec((bm, D_IN), row)]
    for a in w_args:
        in_specs.append(pl.BlockSpec(a.shape, fixed))
    if write_x:
        out_shape = [jax.ShapeDtypeStruct((M, D_MAIN), jnp.float32),
                     jax.ShapeDtypeStruct((M, D_MAIN), jnp.float32)]
        out_specs = [pl.BlockSpec((bm, D_MAIN), row),
                     pl.BlockSpec((bm, D_MAIN), row)]
    else:
        out_shape = jax.ShapeDtypeStruct((M, D_MAIN), jnp.float32)
        out_specs = pl.BlockSpec((bm, D_MAIN), row)
    return pl.pallas_call(
        functools.partial(_enc_body, write_x),
        grid=grid,
        in_specs=in_specs,
        out_specs=out_specs,
        out_shape=out_shape,
    )(xin, *w_args)


def _sims_body(kq_ref, ck_ref, out_ref):
    ck = ck_ref[...]
    csq = jnp.sum(ck * ck, axis=1)
    d = _dot(kq_ref[...], ck)
    out_ref[...] = 2.0 * d - csq[None, :]


def _sims(kq, ck, bq, bc):
    B = kq.shape[0]
    N = ck.shape[0]
    return pl.pallas_call(
        _sims_body,
        grid=(B // bq, N // bc),
        in_specs=[pl.BlockSpec((bq, D_MAIN), lambda i, j: (i, 0)),
                  pl.BlockSpec((bc, D_MAIN), lambda i, j: (j, 0))],
        out_specs=pl.BlockSpec((bq, bc), lambda i, j: (i, j)),
        out_shape=jax.ShapeDtypeStruct((B, N), jnp.float32),
    )(kq, ck)


def _tail_body(vals_ref, ctxk_ref, ctxy_ref, kq_ref, x_ref,
               wle_ref, ble_ref, wta_ref, bta_ref, wtb_ref,
               ga_ref, ba_ref, waa_ref, baa_ref, wab_ref, bab_ref,
               gb_ref, bb_ref, wba_ref, bba_ref, wbb_ref, bbb_ref,
               gh_ref, bh_ref, wh_ref, bhl_ref, out_ref):
    dot = _dot
    vals = vals_ref[...]
    m = jnp.max(vals, axis=-1, keepdims=True)
    e = jnp.exp(vals - m)
    probs = e / jnp.sum(e, axis=-1, keepdims=True)
    bq = vals.shape[0]
    kq = kq_ref[...]
    ctxk = ctxk_ref[...].reshape(bq, CTX, D_MAIN)
    u = (kq[:, None, :] - ctxk).reshape(bq * CTX, D_MAIN)
    t = dot(jnp.maximum(dot(u, wta_ref[...]) + bta_ref[...], 0.0),
            wtb_ref[...]).reshape(bq, CTX, D_MAIN)
    wle_b = jnp.broadcast_to(wle_ref[...], (bq, D_MAIN))
    ble_b = jnp.broadcast_to(ble_ref[...], (bq, D_MAIN))
    ctxy = ctxy_ref[...]
    probs_b = probs.astype(jnp.bfloat16).astype(jnp.float32)
    acc = jnp.zeros((bq, D_MAIN), jnp.float32)
    for c in range(CTX):
        v_c = ctxy[:, c:c + 1] * wle_b + ble_b + t[:, c, :]
        v_c = v_c.astype(jnp.bfloat16).astype(jnp.float32)
        acc = acc + probs_b[:, c:c + 1] * v_c
    x = x_ref[...] + acc
    for g, b, wa, ba, wb, bb in (
            (ga_ref, ba_ref, waa_ref, baa_ref, wab_ref, bab_ref),
            (gb_ref, bb_ref, wba_ref, bba_ref, wbb_ref, bbb_ref)):
        z = _ln(x, g[...], b[...])
        h = dot(jnp.maximum(dot(z, wa[...]) + ba[...], 0.0),
                wb[...]) + bb[...]
        x = x + h
    z = _ln(x, gh_ref[...], bh_ref[...])
    out_ref[...] = dot(jnp.maximum(z, 0.0), wh_ref[...]) + bhl_ref[...]


def _tail_head_pad(p):
    wh = jnp.zeros((128, D_MAIN), jnp.float32).at[0].set(p['WH'][0])
    bhl = jnp.zeros((1, 128), jnp.float32).at[0, 0].set(p['bH_lin'][0])
    return wh, bhl


def _tail(vals, ctxk2d, ctxy, kq, x, p, bq):
    B = vals.shape[0]
    row = lambda i: (i, 0)
    fixed = lambda i: (0, 0)
    wh_pad, bhl_pad = _tail_head_pad(p)
    w_args = [p['W_le'].reshape(1, D_MAIN), p['b_le'].reshape(1, -1),
              p['WTa'], p['bTa'].reshape(1, -1), p['WTb'],
              p['gA'].reshape(1, -1), p['bA'].reshape(1, -1),
              p['WAa'], p['bAa'].reshape(1, -1), p['WAb'], p['bAb'].reshape(1, -1),
              p['gB'].reshape(1, -1), p['bB'].reshape(1, -1),
              p['WBa'], p['bBa'].reshape(1, -1), p['WBb'], p['bBb'].reshape(1, -1),
              p['gH'].reshape(1, -1), p['bH'].reshape(1, -1),
              wh_pad, bhl_pad]
    in_specs = [pl.BlockSpec((bq, CTX), row),
                pl.BlockSpec((bq * CTX, D_MAIN), row),
                pl.BlockSpec((bq, CTX), row),
                pl.BlockSpec((bq, D_MAIN), row),
                pl.BlockSpec((bq, D_MAIN), row)]
    for a in w_args:
        in_specs.append(pl.BlockSpec(a.shape, fixed))
    return pl.pallas_call(
        _tail_body,
        grid=(B // bq,),
        in_specs=in_specs,
        out_specs=pl.BlockSpec((bq, 128), row),
        out_shape=jax.ShapeDtypeStruct((B, 128), jnp.float32),
    )(vals, ctxk2d, ctxy, kq, x, *w_args)[:, :1]


def _topk_gather_sc(sims, ck, y):
    B, N = sims.shape
    NC, QPW = 2, B // 32
    NCH = N // 16
    mesh = plsc.VectorSubcoreMesh(core_axis_name="c", subcore_axis_name="s")

    @functools.partial(
        pl.kernel, mesh=mesh,
        compiler_params=pltpu.CompilerParams(needs_layout_passes=False),
        out_type=[jax.ShapeDtypeStruct((B, CTX), jnp.float32),
                  jax.ShapeDtypeStruct((B * CTX, D_MAIN), jnp.float32),
                  jax.ShapeDtypeStruct((B, CTX), jnp.float32)],
        scratch_types=[
            pltpu.VMEM((N,), jnp.float32),
            pltpu.VMEM((N,), jnp.float32),
            pltpu.VMEM((N + 16,), jnp.int32),
            pltpu.VMEM((N + 16,), jnp.int32),
            pltpu.VMEM((4096,), jnp.int32),
            pltpu.VMEM((48,), jnp.int32),
            pltpu.VMEM((CTX,), jnp.int32),
            pltpu.VMEM((CTX, D_MAIN), jnp.float32),
            pltpu.VMEM((QPW, CTX), jnp.float32),
            pltpu.VMEM((QPW, CTX), jnp.float32),
            pltpu.SemaphoreType.DMA,
        ])
    def body(sims_hbm, ck_hbm, y_hbm, vals_out, ctxk_out, ctxy_out,
             row_v, y_v, eqA, eqB, hist, res48, res32, rows_v,
             vals_all, ctxy_all, sem):
        wid = lax.axis_index("s") * NC + lax.axis_index("c")
        pltpu.sync_copy(y_hbm, y_v)
        iota = lax.iota(jnp.int32, 16)
        ones_i = jnp.ones((16,), jnp.int32)
        zeros_i = jnp.zeros((16,), jnp.int32)
        MSB = jnp.int32(-2147483648)
        POSM = jnp.int32(2147483647)

        def flip(vf32):
            u = lax.bitcast_convert_type(vf32, jnp.int32)
            return jnp.where(u < 0, u ^ POSM, u)

        def hist_clear(nbins):
            def f(bb, _):
                hist[pl.ds(bb * 16, 16)] = zeros_i
                return 0
            lax.fori_loop(0, nbins, f, 0)

        def scan_bins(nbins, need):
            def cond(st):
                bb, cum = st
                return jnp.logical_and(bb >= 0, cum < need)

            def step(st):
                bb, cum = st
                t = jnp.sum(hist[pl.ds(bb * 16, 16)])
                return bb - 1, cum + t
            bend, cum = lax.while_loop(cond, step, (jnp.int32(nbins - 1),
                                                    jnp.int32(0)))
            bstar = jnp.minimum(bend + 1, jnp.int32(nbins - 1))
            tstar = jnp.sum(hist[pl.ds(bstar * 16, 16)])
            above = cum - tstar
            return bstar, above

        def process_query(j, _):
            q = wid * QPW + j
            pltpu.sync_copy(sims_hbm.at[q], row_v)
            hist_clear(256)

            def h0(ch, _):
                s = flip(row_v[pl.ds(ch * 16, 16)])
                b0 = lax.shift_right_logical(s ^ MSB, 24)
                plsc.addupdate_scatter(hist, [b0 * 16 + iota], ones_i)
                return 0
            lax.fori_loop(0, NCH, h0, 0)
            bstar, _ = scan_bins(256, jnp.int32(CTX))

            def c0(ch, st):
                aoff, eoff = st
                s = flip(row_v[pl.ds(ch * 16, 16)])
                b0 = lax.shift_right_logical(s ^ MSB, 24)
                idx = iota + ch * 16
                mgt = b0 > bstar
                meq = b0 == bstar
                plsc.store_compressed(res48.at[pl.ds(aoff, 16)], idx,
                                      mask=mgt)
                plsc.store_compressed(eqA.at[pl.ds(eoff, 16)], idx,
                                      mask=meq)
                return (aoff + jnp.sum(mgt.astype(jnp.int32)),
                        eoff + jnp.sum(meq.astype(jnp.int32)))
            aoff, eq_cnt = lax.fori_loop(0, NCH, c0,
                                         (jnp.int32(0), jnp.int32(0)))
            need = jnp.int32(CTX) - aoff

            bufs = (eqA, eqB)
            for lev, sh in enumerate((20, 16, 12, 8, 4, 0)):
                src, dst = bufs[lev % 2], bufs[(lev + 1) % 2]
                hist_clear(16)
                nch = jnp.where(need > 0, (eq_cnt + 15) // 16, 0)

                def h(ch, _, src=src, sh=sh, eq_cnt=eq_cnt):
                    rem = eq_cnt - ch * 16
                    m = iota < rem
                    idxv = src[pl.ds(ch * 16, 16)]
                    s = flip(plsc.load_gather(row_v, [idxv], mask=m))
                    bb = lax.shift_right_logical(s, sh) & 15
                    plsc.addupdate_scatter(hist, [bb * 16 + iota], ones_i, mask=m)
                    return 0
                lax.fori_loop(0, nch, h, 0)
                bstar, _ = scan_bins(16, need)

                def c(ch, st, src=src, dst=dst, sh=sh, eq_cnt=eq_cnt,
                      bstar=bstar):
                    aoff, eoff = st
                    rem = eq_cnt - ch * 16
                    m = iota < rem
                    idxv = src[pl.ds(ch * 16, 16)]
                    s = flip(plsc.load_gather(row_v, [idxv], mask=m))
                    bb = lax.shift_right_logical(s, sh) & 15
                    mgt = m & (bb > bstar)
                    meq = m & (bb == bstar)
                    plsc.store_compressed(res48.at[pl.ds(aoff, 16)], idxv,
                                          mask=mgt)
                    plsc.store_compressed(dst.at[pl.ds(eoff, 16)], idxv,
                                          mask=meq)
                    return (aoff + jnp.sum(mgt.astype(jnp.int32)),
                            eoff + jnp.sum(meq.astype(jnp.int32)))
                aoff, eq_cnt = lax.fori_loop(0, nch, c,
                                             (aoff, jnp.int32(0)))
                need = jnp.int32(CTX) - aoff

            fin = bufs[0]

            def t(ch, roff, fin=fin):
                rem = need - ch * 16
                m = iota < rem
                idxv = fin[pl.ds(ch * 16, 16)]
                plsc.store_compressed(res48.at[pl.ds(roff, 16)], idxv,
                                      mask=m)
                return roff + jnp.sum(m.astype(jnp.int32))
            lax.fori_loop(0, (need + 15) // 16, t, aoff)

            res32[pl.ds(0, 16)] = res48[pl.ds(0, 16)]
            res32[pl.ds(16, 16)] = res48[pl.ds(16, 16)]

            pltpu.async_copy(ck_hbm.at[res32], rows_v, sem).wait()
            pltpu.sync_copy(rows_v, ctxk_out.at[pl.ds(q * CTX, CTX)])
            i0 = res32[pl.ds(0, 16)]
            i1 = res32[pl.ds(16, 16)]
            vals_all[j, pl.ds(0, 16)] = plsc.load_gather(row_v, [i0])
            vals_all[j, pl.ds(16, 16)] = plsc.load_gather(row_v, [i1])
            ctxy_all[j, pl.ds(0, 16)] = plsc.load_gather(y_v, [i0])
            ctxy_all[j, pl.ds(16, 16)] = plsc.load_gather(y_v, [i1])
            return 0

        lax.fori_loop(0, QPW, process_query, 0)
        pltpu.sync_copy(vals_all, vals_out.at[pl.ds(wid * QPW, QPW)])
        pltpu.sync_copy(ctxy_all, ctxy_out.at[pl.ds(wid * QPW, QPW)])

    return body(sims, ck, y)


def kernel(x_num, candidate_x_num, candidate_y, params, context_size):
    p = params
    B = x_num.shape[0]
    N = candidate_x_num.shape[0]
    ck = _encode(candidate_x_num, p, write_x=False, bm=2048)
    xq, kq = _encode(x_num, p, write_x=True, bm=1024)
    sims = _sims(kq, ck, bq=256, bc=2048)
    vals, ctxk2d, ctxy = _topk_gather_sc(sims, ck, candidate_y)
    out = _tail(vals, ctxk2d, ctxy, kq, xq, p, bq=256)
    return out
